# baseline (speedup 1.0000x reference)
import jax
import jax.numpy as jnp
from jax.experimental import pallas as pl

IN_CHANNELS = 384
NUM_EXPERTS = 16
TEMPERATURE = 0.5
HW = 28 * 28
BATCH = 64
B_BLK = 8


def _router_kernel(x_ref, wt_ref, b_ref, o_ref):
    s = jnp.sum(x_ref[...], axis=-1)
    logits = jax.lax.dot_general(
        s, wt_ref[...], (((1,), (0,)), ((), ())),
        preferred_element_type=jnp.float32,
    ) + b_ref[...]
    m = jnp.max(logits, axis=-1, keepdims=True)
    e = jnp.exp(logits - m)
    o_ref[...] = e / jnp.sum(e, axis=-1, keepdims=True)


def kernel(x, W, b):
    xr = x.reshape(BATCH, IN_CHANNELS, HW)
    wt = (W.T / (HW * TEMPERATURE)).astype(jnp.float32)
    b2 = (b / TEMPERATURE).reshape(1, NUM_EXPERTS).astype(jnp.float32)
    grid = (BATCH // B_BLK,)
    out = pl.pallas_call(
        _router_kernel,
        grid=grid,
        in_specs=[
            pl.BlockSpec((B_BLK, IN_CHANNELS, HW), lambda i: (i, 0, 0)),
            pl.BlockSpec((IN_CHANNELS, NUM_EXPERTS), lambda i: (0, 0)),
            pl.BlockSpec((1, NUM_EXPERTS), lambda i: (0, 0)),
        ],
        out_specs=pl.BlockSpec((B_BLK, NUM_EXPERTS), lambda i: (i, 0)),
        out_shape=jax.ShapeDtypeStruct((BATCH, NUM_EXPERTS), jnp.float32),
    )(xr, wt, b2)
    return out
